# baseline (speedup 1.0000x reference)
import jax
import jax.numpy as jnp
from jax import lax
from jax.experimental import pallas as pl
from jax.experimental.pallas import tpu as pltpu
from jax.experimental.pallas import tpu_sc as plsc

SUB = 128
K = 8
CHUNK = SUB * K


def _build(N, D):
    info = plsc.get_sparse_core_info()
    nw = info.num_cores * info.num_subcores
    n_per_w = N // nw
    nchunk = n_per_w // CHUNK
    mesh = plsc.VectorSubcoreMesh(core_axis_name="c", subcore_axis_name="s")

    def body(idx_hbm, table_hbm, out_hbm, idx_v, rows_v, sem):
        wid = lax.axis_index("s") * info.num_cores + lax.axis_index("c")
        base = wid * n_per_w

        def step(g, carry):
            start = base + g * CHUNK
            row0 = pl.multiple_of(start // SUB, 8)
            pltpu.sync_copy(idx_hbm.at[pl.ds(row0, K)], idx_v)
            copies = [
                pltpu.async_copy(
                    table_hbm.at[idx_v.at[j]],
                    rows_v.at[pl.ds(j * SUB, SUB)],
                    sem,
                )
                for j in range(K)
            ]
            for cp in copies:
                cp.wait()
            pltpu.sync_copy(rows_v, out_hbm.at[pl.ds(start, CHUNK)])
            return carry

        lax.fori_loop(0, nchunk, step, 0)

    return pl.kernel(
        body,
        out_type=jax.ShapeDtypeStruct((N, D), jnp.float32),
        mesh=mesh,
        scratch_types=[
            pltpu.VMEM((K, SUB), jnp.int32),
            pltpu.VMEM((CHUNK, D), jnp.float32),
            pltpu.SemaphoreType.DMA,
        ],
        compiler_params=pltpu.CompilerParams(use_tc_tiling_on_sc=False),
    )


def kernel(x, weight):
    B, S = x.shape
    V, D = weight.shape
    N = B * S
    idx2d = x.reshape(N // SUB, SUB).astype(jnp.int32)
    out = _build(N, D)(idx2d, weight)
    return out.reshape(B, S, D)


# speedup vs baseline: 1.0009x; 1.0009x over previous
import jax
import jax.numpy as jnp
from jax import lax
from jax.experimental import pallas as pl
from jax.experimental.pallas import tpu as pltpu
from jax.experimental.pallas import tpu_sc as plsc

CHUNK = 1024


def _build(N, D):
    info = plsc.get_sparse_core_info()
    nw = info.num_cores * info.num_subcores
    n_per_w = N // nw
    nchunk = n_per_w // CHUNK
    mesh = plsc.VectorSubcoreMesh(core_axis_name="c", subcore_axis_name="s")

    def body(idx_hbm, table_hbm, out_hbm, idx_v, rows_v, sem):
        wid = lax.axis_index("s") * info.num_cores + lax.axis_index("c")
        base = wid * n_per_w

        def step(g, carry):
            start = base + g * CHUNK
            pltpu.sync_copy(idx_hbm.at[pl.ds(start, CHUNK)], idx_v)
            pltpu.async_copy(table_hbm.at[idx_v], rows_v, sem).wait()
            pltpu.sync_copy(rows_v, out_hbm.at[pl.ds(start, CHUNK)])
            return carry

        lax.fori_loop(0, nchunk, step, 0)

    return pl.kernel(
        body,
        out_type=jax.ShapeDtypeStruct((N, D), jnp.float32),
        mesh=mesh,
        scratch_types=[
            pltpu.VMEM((CHUNK,), jnp.int32),
            pltpu.VMEM((CHUNK, D), jnp.float32),
            pltpu.SemaphoreType.DMA,
        ],
        compiler_params=pltpu.CompilerParams(use_tc_tiling_on_sc=False),
    )


def kernel(x, weight):
    B, S = x.shape
    V, D = weight.shape
    N = B * S
    idx_flat = x.reshape(N).astype(jnp.int32)
    out = _build(N, D)(idx_flat, weight)
    return out.reshape(B, S, D)


# speedup vs baseline: 1.0143x; 1.0134x over previous
import jax
import jax.numpy as jnp
from jax import lax
from jax.experimental import pallas as pl
from jax.experimental.pallas import tpu as pltpu
from jax.experimental.pallas import tpu_sc as plsc

CHUNK = 800


def _build(N, D):
    info = plsc.get_sparse_core_info()
    nw = info.num_cores * info.num_subcores
    n_per_w = N // nw
    nchunk = n_per_w // CHUNK
    assert n_per_w % CHUNK == 0 and nchunk % 2 == 0 and nchunk >= 4
    mesh = plsc.VectorSubcoreMesh(core_axis_name="c", subcore_axis_name="s")

    def body(idx_hbm, table_hbm, out_hbm, idx0, idx1, rows0, rows1,
             sg0, sg1, so0, so1):
        wid = lax.axis_index("s") * info.num_cores + lax.axis_index("c")
        base = wid * n_per_w
        idxv, rowsv = (idx0, idx1), (rows0, rows1)
        sg, so = (sg0, sg1), (so0, so1)

        def hbm_slice(hbm, c):
            return hbm.at[pl.ds(base + c * CHUNK, CHUNK)]

        def load_and_fire(c, b):
            pltpu.sync_copy(hbm_slice(idx_hbm, c), idxv[b])
            pltpu.async_copy(table_hbm.at[idxv[b]], rowsv[b], sg[b])

        def wait_gather(b):
            pltpu.make_async_copy(
                table_hbm.at[pl.ds(0, CHUNK)], rowsv[b], sg[b]).wait()

        def fire_out(c, b):
            pltpu.async_copy(rowsv[b], hbm_slice(out_hbm, c), so[b])

        def wait_out(b):
            pltpu.make_async_copy(
                rowsv[b], hbm_slice(out_hbm, 0), so[b]).wait()

        load_and_fire(0, 0)
        load_and_fire(1, 1)
        wait_gather(0)
        fire_out(0, 0)

        def step(t, carry):
            for b in range(2):
                c = 2 * t + b
                wait_out(b)
                load_and_fire(c, b)
                wait_gather(1 - b)
                fire_out(c - 1, 1 - b)
            return carry

        lax.fori_loop(1, nchunk // 2, step, 0)

        wait_gather(1)
        fire_out(nchunk - 1, 1)
        wait_out(0)
        wait_out(1)

    return pl.kernel(
        body,
        out_type=jax.ShapeDtypeStruct((N, D), jnp.float32),
        mesh=mesh,
        scratch_types=[
            pltpu.VMEM((CHUNK,), jnp.int32),
            pltpu.VMEM((CHUNK,), jnp.int32),
            pltpu.VMEM((CHUNK, D), jnp.float32),
            pltpu.VMEM((CHUNK, D), jnp.float32),
            pltpu.SemaphoreType.DMA,
            pltpu.SemaphoreType.DMA,
            pltpu.SemaphoreType.DMA,
            pltpu.SemaphoreType.DMA,
        ],
        compiler_params=pltpu.CompilerParams(use_tc_tiling_on_sc=False),
    )


def kernel(x, weight):
    B, S = x.shape
    V, D = weight.shape
    N = B * S
    idx_flat = x.reshape(N).astype(jnp.int32)
    out = _build(N, D)(idx_flat, weight)
    return out.reshape(B, S, D)
